# Initial kernel scaffold; baseline (speedup 1.0000x reference)
import jax
import jax.numpy as jnp
from jax.experimental import pallas as pl


def kernel(user_emb, entity_emb, latent_emb, edge_index, edge_type, interact_row, interact_col, interact_val, weight, disen_weight_att):
    raise NotImplementedError("write your pallas kernel here")


# speedup vs baseline: 2.6363x; 2.6363x over previous
import functools

import jax
import jax.numpy as jnp
from jax import lax
from jax.experimental import pallas as pl
from jax.experimental.pallas import tpu as pltpu
from jax.experimental.pallas import tpu_sc as plsc

N_ENT = 10000
N_USR = 20000
CH = 128
CHH = 64
N_FAC = 4
N_REL = 12
E = 320000
NNZ = 500000

NC = 2
NS = 16
LANES = 16

C = 128
SUP = 8
SUPE = C * SUP

EPT = 20480
EPAD = EPT * NS
ESUP = EPT // SUPE

NPT = 31744
NPAD = NPT * NS
NSUP = NPT // SUPE

ENT_PT = N_ENT // NS
USR_PT = N_USR // NS
ZROWS = 25


def _splat(v16, i):
  return v16.at[jnp.full((16,), i, jnp.int32)].get(mode="promise_in_bounds")


def _sc_body(t_lo, t_hi, tail_p, typ_p, head2d, col_p, val_p, row2d, rel2,
             ent_out, usr_out, cnt_out,
             acc, cacc, rows_v, idx_s, typ_s, head_sv, val_s, relv, onesv,
             zbuf, zbuf16, sem):
  cid = lax.axis_index("c")
  sid = lax.axis_index("s")

  zero16 = jnp.zeros((16,), jnp.float32)
  one16 = jnp.ones((16,), jnp.float32)
  colc = [lax.iota(jnp.int32, 16) + 16 * j for j in range(4)]

  def _fill_zb(i, carry):
    for j in range(4):
      zbuf[i, 16 * j:16 * (j + 1)] = zero16
    zbuf16[i, :] = zero16
    return carry

  lax.fori_loop(0, ZROWS, _fill_zb, 0)

  def _fill_ones(i, carry):
    onesv[i, :] = one16
    return carry

  lax.fori_loop(0, C, _fill_ones, 0)

  def _work(table, half, do_cnt):
    pltpu.sync_copy(rel2.at[half], relv)

    for k in range(ENT_PT // ZROWS):
      pltpu.sync_copy(zbuf, acc.at[pl.ds(sid * ENT_PT + k * ZROWS, ZROWS)])
    if do_cnt:
      for k in range(ENT_PT // ZROWS):
        pltpu.sync_copy(zbuf16, cacc.at[pl.ds(sid * ENT_PT + k * ZROWS, ZROWS)])
    plsc.subcore_barrier()

    ebase = sid * EPT
    erow = sid * (EPT // C)

    def _esup(s, carry):
      eoff = pl.multiple_of(ebase + s * SUPE, SUPE)
      pltpu.sync_copy(tail_p.at[pl.ds(eoff, SUPE)], idx_s)
      pltpu.sync_copy(typ_p.at[pl.ds(eoff, SUPE)], typ_s)
      pltpu.sync_copy(head2d.at[pl.ds(pl.multiple_of(erow + s * SUP, SUP),
                                      SUP)], head_sv)

      def _echunk(c8, carry2):
        pltpu.async_copy(table.at[idx_s.at[pl.ds(c8 * C, C)]], rows_v,
                         sem).wait()

        def _egrp(g, carry3):
          t16 = typ_s[pl.ds(c8 * C + g * 16, 16)]
          r16 = jnp.where(t16 == 0, 10, t16 - 1) * 64
          for i in range(16):
            rb = _splat(r16, i)
            gi = g * 16 + i
            for j in range(4):
              rel_j = plsc.load_gather(relv, [rb + colc[j]])
              rows_v[gi, 16 * j:16 * (j + 1)] = (
                  rows_v[gi, 16 * j:16 * (j + 1)] * rel_j)
          return carry3

        lax.fori_loop(0, SUP, _egrp, 0)
        pltpu.sync_copy(rows_v, acc.at[head_sv.at[c8]], add=True)
        if do_cnt:
          pltpu.sync_copy(onesv, cacc.at[head_sv.at[c8]], add=True)
        return carry2

      lax.fori_loop(0, SUP, _echunk, 0)
      return carry

    lax.fori_loop(0, ESUP, _esup, 0)
    plsc.subcore_barrier()

    pltpu.sync_copy(acc.at[pl.ds(sid * ENT_PT, ENT_PT)],
                    ent_out.at[half, sid])
    if do_cnt:
      pltpu.sync_copy(cacc.at[pl.ds(sid * ENT_PT, ENT_PT)],
                      cnt_out.at[sid])
    plsc.subcore_barrier()

    for k in range(USR_PT // ZROWS):
      pltpu.sync_copy(zbuf, acc.at[pl.ds(sid * USR_PT + k * ZROWS, ZROWS)])
    plsc.subcore_barrier()

    nbase = sid * NPT
    nrow = sid * (NPT // C)

    def _usup(s, carry):
      noff = pl.multiple_of(nbase + s * SUPE, SUPE)
      pltpu.sync_copy(col_p.at[pl.ds(noff, SUPE)], idx_s)
      pltpu.sync_copy(val_p.at[pl.ds(noff, SUPE)], val_s)
      pltpu.sync_copy(row2d.at[pl.ds(pl.multiple_of(nrow + s * SUP, SUP),
                                     SUP)], head_sv)

      def _uchunk(c8, carry2):
        pltpu.async_copy(table.at[idx_s.at[pl.ds(c8 * C, C)]], rows_v,
                         sem).wait()

        def _ugrp(g, carry3):
          v16 = val_s[pl.ds(c8 * C + g * 16, 16)]
          for i in range(16):
            vb = _splat(v16, i)
            gi = g * 16 + i
            for j in range(4):
              rows_v[gi, 16 * j:16 * (j + 1)] = (
                  rows_v[gi, 16 * j:16 * (j + 1)] * vb)
          return carry3

        lax.fori_loop(0, SUP, _ugrp, 0)
        pltpu.sync_copy(rows_v, acc.at[head_sv.at[c8]], add=True)
        return carry2

      lax.fori_loop(0, SUP, _uchunk, 0)
      return carry

    lax.fori_loop(0, NSUP, _usup, 0)
    plsc.subcore_barrier()

    pltpu.sync_copy(acc.at[pl.ds(sid * USR_PT, USR_PT)],
                    usr_out.at[half, sid])

  @pl.when(cid == 0)
  def _():
    _work(t_lo, 0, True)

  @pl.when(cid == 1)
  def _():
    _work(t_hi, 1, False)


@functools.cache
def _sc_call():
  mesh = plsc.VectorSubcoreMesh(core_axis_name="c", subcore_axis_name="s",
                                num_cores=NC, num_subcores=NS)
  return pl.kernel(
      _sc_body,
      out_type=(
          jax.ShapeDtypeStruct((NC, NS, ENT_PT, CHH), jnp.float32),
          jax.ShapeDtypeStruct((NC, NS, USR_PT, CHH), jnp.float32),
          jax.ShapeDtypeStruct((NS, ENT_PT, 16), jnp.float32),
      ),
      mesh=mesh,
      compiler_params=pltpu.CompilerParams(needs_layout_passes=False,
                                           use_tc_tiling_on_sc=False),
      scratch_types=[
          pltpu.VMEM_SHARED((N_USR, CHH), jnp.float32),
          pltpu.VMEM_SHARED((N_ENT, 16), jnp.float32),
          pltpu.VMEM((C, CHH), jnp.float32),
          pltpu.VMEM((SUPE,), jnp.int32),
          pltpu.VMEM((SUPE,), jnp.int32),
          pltpu.VMEM((SUP, C), jnp.int32),
          pltpu.VMEM((SUPE,), jnp.float32),
          pltpu.VMEM((768,), jnp.float32),
          pltpu.VMEM((C, 16), jnp.float32),
          pltpu.VMEM((ZROWS, CHH), jnp.float32),
          pltpu.VMEM((ZROWS, 16), jnp.float32),
          pltpu.SemaphoreType.DMA,
      ],
  )


_ENT_BLK = 2000
_USR_BLK = 2000


def _ent_dense_body(s0, s1, cnt, res_in, e0_o, e1_o, res_o):
  s = jnp.concatenate([s0[...], s1[...]], axis=1)
  d = s / jnp.maximum(cnt[...], 1.0)
  n = jnp.sqrt(jnp.sum(d * d, axis=1, keepdims=True))
  e = d / jnp.maximum(n, 1e-12)
  e0_o[...] = e[:, :CHH]
  e1_o[...] = e[:, CHH:]
  res_o[...] = res_in[...] + e


@jax.jit
def _ent_dense(s0, s1, cnt1, res_in):
  g = N_ENT // _ENT_BLK
  return pl.pallas_call(
      _ent_dense_body,
      grid=(g,),
      in_specs=[
          pl.BlockSpec((_ENT_BLK, CHH), lambda i: (i, 0)),
          pl.BlockSpec((_ENT_BLK, CHH), lambda i: (i, 0)),
          pl.BlockSpec((_ENT_BLK, 1), lambda i: (i, 0)),
          pl.BlockSpec((_ENT_BLK, CH), lambda i: (i, 0)),
      ],
      out_specs=[
          pl.BlockSpec((_ENT_BLK, CHH), lambda i: (i, 0)),
          pl.BlockSpec((_ENT_BLK, CHH), lambda i: (i, 0)),
          pl.BlockSpec((_ENT_BLK, CH), lambda i: (i, 0)),
      ],
      out_shape=[
          jax.ShapeDtypeStruct((N_ENT, CHH), jnp.float32),
          jax.ShapeDtypeStruct((N_ENT, CHH), jnp.float32),
          jax.ShapeDtypeStruct((N_ENT, CH), jnp.float32),
      ],
  )(s0, s1, cnt1, res_in)


def _usr_dense_body(s0, s1, up, res_in, lat, dw, u_o, res_o):
  z = lax.dot_general(up[...], lat[...], (((1,), (1,)), ((), ())),
                      preferred_element_type=jnp.float32)
  z = z - jnp.max(z, axis=1, keepdims=True)
  ez = jnp.exp(z)
  score = ez / jnp.sum(ez, axis=1, keepdims=True)
  gate = 1.0 + lax.dot_general(score, dw[...], (((1,), (0,)), ((), ())),
                               preferred_element_type=jnp.float32)
  x = jnp.concatenate([s0[...], s1[...]], axis=1) * gate
  n = jnp.sqrt(jnp.sum(x * x, axis=1, keepdims=True))
  u = x / jnp.maximum(n, 1e-12)
  u_o[...] = u
  res_o[...] = res_in[...] + u


@jax.jit
def _usr_dense(s0, s1, up, res_in, lat, dw):
  g = N_USR // _USR_BLK
  return pl.pallas_call(
      _usr_dense_body,
      grid=(g,),
      in_specs=[
          pl.BlockSpec((_USR_BLK, CHH), lambda i: (i, 0)),
          pl.BlockSpec((_USR_BLK, CHH), lambda i: (i, 0)),
          pl.BlockSpec((_USR_BLK, CH), lambda i: (i, 0)),
          pl.BlockSpec((_USR_BLK, CH), lambda i: (i, 0)),
          pl.BlockSpec((N_FAC, CH), lambda i: (0, 0)),
          pl.BlockSpec((N_FAC, CH), lambda i: (0, 0)),
      ],
      out_specs=[
          pl.BlockSpec((_USR_BLK, CH), lambda i: (i, 0)),
          pl.BlockSpec((_USR_BLK, CH), lambda i: (i, 0)),
      ],
      out_shape=[
          jax.ShapeDtypeStruct((N_USR, CH), jnp.float32),
          jax.ShapeDtypeStruct((N_USR, CH), jnp.float32),
      ],
  )(s0, s1, up, res_in, lat, dw)


def _cor_body(w_ref, da_ref, dw_o, cor_o):
  da = da_ref[...]
  m = jnp.max(da, axis=1, keepdims=True)
  ez = jnp.exp(da - m)
  sm = ez / jnp.sum(ez, axis=1, keepdims=True)
  dw_o[...] = lax.dot_general(sm, w_ref[...], (((1,), (0,)), ((), ())),
                              preferred_element_type=jnp.float32)
  nrm = jnp.sqrt(jnp.sum(da * da, axis=1, keepdims=True))
  nd = da / nrm
  gram = lax.dot_general(nd, nd, (((1,), (1,)), ((), ())),
                         preferred_element_type=jnp.float32)
  ii = lax.broadcasted_iota(jnp.int32, (N_FAC, N_FAC), 0)
  jj = lax.broadcasted_iota(jnp.int32, (N_FAC, N_FAC), 1)
  cor_o[...] = jnp.sum(jnp.where(ii < jj, gram * gram, 0.0),
                       keepdims=True).reshape(1, 1)


@jax.jit
def _cor_call(weight, disen_weight_att):
  return pl.pallas_call(
      _cor_body,
      out_shape=[
          jax.ShapeDtypeStruct((N_FAC, CH), jnp.float32),
          jax.ShapeDtypeStruct((1, 1), jnp.float32),
      ],
  )(weight, disen_weight_att)


def kernel(user_emb, entity_emb, latent_emb, edge_index, edge_type,
           interact_row, interact_col, interact_val, weight,
           disen_weight_att):
  head = edge_index[0]
  tail = edge_index[1]
  tail_p = jnp.pad(tail, (0, EPAD - E))
  typ_p = jnp.pad(edge_type, (0, EPAD - E), constant_values=N_REL)
  head2d = jnp.pad(head, (0, EPAD - E)).reshape(-1, C)
  col_p = jnp.pad(interact_col, (0, NPAD - NNZ))
  val_p = jnp.pad(interact_val, (0, NPAD - NNZ))
  row2d = jnp.pad(interact_row, (0, NPAD - NNZ)).reshape(-1, C)
  w12 = jnp.concatenate([weight, jnp.zeros((1, CH), jnp.float32)], axis=0)
  rel2 = w12.reshape(N_REL, NC, CHH).transpose(1, 0, 2).reshape(NC, N_REL * CHH)

  dw, cor = _cor_call(weight, disen_weight_att)

  sc = _sc_call()
  ent_res, usr_res = entity_emb, user_emb
  u = user_emb
  e0 = entity_emb[:, :CHH]
  e1 = entity_emb[:, CHH:]
  cnt1 = None
  for hop in range(2):
    ent_sums, usr_sums, cnt16 = sc(e0, e1, tail_p, typ_p, head2d, col_p,
                                   val_p, row2d, rel2)
    ent_sums = ent_sums.reshape(NC, N_ENT, CHH)
    usr_sums = usr_sums.reshape(NC, N_USR, CHH)
    if hop == 0:
      cnt1 = cnt16.reshape(N_ENT, 16)[:, :1]
    e0, e1, ent_res = _ent_dense(ent_sums[0], ent_sums[1], cnt1, ent_res)
    u, usr_res = _usr_dense(usr_sums[0], usr_sums[1], u, usr_res,
                            latent_emb, dw)
  return ent_res, usr_res, cor.reshape(())


# speedup vs baseline: 2.9667x; 1.1253x over previous
import functools

import jax
import jax.numpy as jnp
from jax import lax
from jax.experimental import pallas as pl
from jax.experimental.pallas import tpu as pltpu
from jax.experimental.pallas import tpu_sc as plsc

N_ENT = 10000
N_USR = 20000
CH = 128
CHH = 64
N_FAC = 4
N_REL = 12
E = 320000
NNZ = 500000

NC = 2
NS = 16
NB = 3

C = 128
SCH = 6
SUPE = SCH * C

ENSUP = 28
EPT = ENSUP * SUPE
EPAD = EPT * NS

UNSUP = 42
NPT = UNSUP * SUPE
NPAD = NPT * NS

ENT_PT = N_ENT // NS
USR_PT = N_USR // NS
ZROWS = 25


def _splat(v16, i):
  return v16.at[jnp.full((16,), i, jnp.int32)].get(mode="promise_in_bounds")


def _sc_body(do_cnt, t_lo, t_hi, tail4, typ4, head4, col4, val4, row4, rel2,
             ent_out, usr_out, cnt_out,
             acc, cacc, rows0, rows1, rows2,
             idxa, idxb, tva, tvb, hra, hrb,
             relv, onesv, zbuf, zbuf16,
             gsem0, gsem1, gsem2, ssem0, ssem1, ssem2, csem, isema, isemb):
  cid = lax.axis_index("c")
  sid = lax.axis_index("s")
  rows = (rows0, rows1, rows2)
  gsem = (gsem0, gsem1, gsem2)
  ssem = (ssem0, ssem1, ssem2)
  idx_st = (idxa, idxb)
  tv_st = (tva, tvb)
  hr_st = (hra, hrb)
  isem = (isema, isemb)

  zero16 = jnp.zeros((16,), jnp.float32)
  one16 = jnp.ones((16,), jnp.float32)
  colc = [lax.iota(jnp.int32, 16) + 16 * j for j in range(4)]

  def _fill_zb(i, carry):
    for j in range(4):
      zbuf[i, 16 * j:16 * (j + 1)] = zero16
    zbuf16[i, :] = zero16
    return carry

  lax.fori_loop(0, ZROWS, _fill_zb, 0)

  def _fill_ones(i, carry):
    onesv[i, :] = one16
    return carry

  lax.fori_loop(0, C, _fill_ones, 0)

  def _work(table, half, with_cnt, arr3, tv3, hr4, nsup, compute, cnt_phase):
    def _stage_issue(s, par):
      pltpu.async_copy(arr3.at[sid, s], idx_st[par], isem[par])
      pltpu.async_copy(tv3.at[sid, s], tv_st[par], isem[par])
      pltpu.async_copy(hr4.at[sid, s], hr_st[par], isem[par])

    def _stage_wait(par):
      pltpu.make_async_copy(arr3.at[sid, 0], idx_st[par], isem[par]).wait()
      pltpu.make_async_copy(tv3.at[sid, 0], tv_st[par], isem[par]).wait()
      pltpu.make_async_copy(hr4.at[sid, 0], hr_st[par], isem[par]).wait()

    def _gissue(par, k, b):
      pltpu.async_copy(table.at[idx_st[par].at[pl.ds(k * C, C)]], rows[b],
                       gsem[b])

    def _gwait(b):
      pltpu.make_async_copy(table.at[idxa.at[pl.ds(0, C)]], rows[b],
                            gsem[b]).wait()

    def _sissue(par, k, b):
      pltpu.async_copy(rows[b], acc.at[hr_st[par].at[k]], ssem[b], add=True)

    def _swait(b):
      pltpu.make_async_copy(rows[b], acc.at[hra.at[0]], ssem[b]).wait()

    def _cwait():
      pltpu.make_async_copy(onesv, cacc.at[hra.at[0]], csem).wait()

    _stage_issue(0, 0)
    _stage_wait(0)
    _gissue(0, 0, 0)
    _gissue(0, 1, 1)

    def _pair(sp, carry):
      for par in range(2):
        s = sp * 2 + par
        nxt = par ^ 1
        for k in range(SCH):
          b = k % NB
          bb = (b + 2) % NB
          _gwait(b)
          compute(par, k, rows[b])
          _sissue(par, k, b)
          if cnt_phase:
            if k == 0:
              @pl.when(s >= 1)
              def _():
                _cwait()
            else:
              _cwait()
            pltpu.async_copy(onesv, cacc.at[hr_st[par].at[k]], csem,
                             add=True)
          if k == 1:
            @pl.when(s + 1 < nsup)
            def _():
              _stage_issue(s + 1, nxt)
          if k < SCH - 2:
            if k == 0:
              @pl.when(s >= 1)
              def _():
                _swait(bb)
            else:
              _swait(bb)
            _gissue(par, k + 2, bb)
          else:
            if k == SCH - 2:
              @pl.when(s + 1 < nsup)
              def _():
                _stage_wait(nxt)
                _swait(bb)
                _gissue(nxt, k - 4, bb)
            else:
              @pl.when(s + 1 < nsup)
              def _():
                _swait(bb)
                _gissue(nxt, k - 4, bb)
      return carry

    lax.fori_loop(0, nsup // 2, _pair, 0)
    for b in range(NB):
      _swait(b)
    if cnt_phase:
      _cwait()

  def _side(table, half, with_cnt):
    pltpu.sync_copy(rel2.at[half], relv)
    for k in range(ENT_PT // ZROWS):
      pltpu.sync_copy(zbuf, acc.at[pl.ds(sid * ENT_PT + k * ZROWS, ZROWS)])
    if with_cnt:
      for k in range(ENT_PT // ZROWS):
        pltpu.sync_copy(zbuf16, cacc.at[pl.ds(sid * ENT_PT + k * ZROWS,
                                              ZROWS)])
    plsc.subcore_barrier()

    def _ecompute(par, k, rbuf):
      base = k * C

      def _egrp(g, carry):
        t16 = tv_st[par][pl.ds(base + g * 16, 16)]
        r16 = jnp.where(t16 == 0, 10, t16 - 1) * 64

        def _edge(i, carry2):
          rb = _splat(r16, i)
          gi = g * 16 + i
          for j in range(4):
            rel_j = plsc.load_gather(relv, [rb + colc[j]])
            rbuf[gi, 16 * j:16 * (j + 1)] = (
                rbuf[gi, 16 * j:16 * (j + 1)] * rel_j)
          return carry2

        lax.fori_loop(0, 16, _edge, 0)
        return carry

      lax.fori_loop(0, C // 16, _egrp, 0)

    _work(table, half, with_cnt, tail4, typ4, head4, ENSUP, _ecompute,
          with_cnt)
    plsc.subcore_barrier()

    pltpu.sync_copy(acc.at[pl.ds(sid * ENT_PT, ENT_PT)],
                    ent_out.at[half, sid])
    if with_cnt:
      pltpu.sync_copy(cacc.at[pl.ds(sid * ENT_PT, ENT_PT)],
                      cnt_out.at[sid])
    plsc.subcore_barrier()
    for k in range(USR_PT // ZROWS):
      pltpu.sync_copy(zbuf, acc.at[pl.ds(sid * USR_PT + k * ZROWS, ZROWS)])
    plsc.subcore_barrier()

    def _ucompute(par, k, rbuf):
      base = k * C

      def _ugrp(g, carry):
        v16 = plsc.bitcast(tv_st[par][pl.ds(base + g * 16, 16)],
                           jnp.float32)

        def _edge(i, carry2):
          vb = _splat(v16, i)
          gi = g * 16 + i
          for j in range(4):
            rbuf[gi, 16 * j:16 * (j + 1)] = (
                rbuf[gi, 16 * j:16 * (j + 1)] * vb)
          return carry2

        lax.fori_loop(0, 16, _edge, 0)
        return carry

      lax.fori_loop(0, C // 16, _ugrp, 0)

    _work(table, half, with_cnt, col4, val4, row4, UNSUP, _ucompute, False)
    plsc.subcore_barrier()

    pltpu.sync_copy(acc.at[pl.ds(sid * USR_PT, USR_PT)],
                    usr_out.at[half, sid])

  @pl.when(cid == 0)
  def _():
    _side(t_lo, 0, do_cnt)

  @pl.when(cid == 1)
  def _():
    _side(t_hi, 1, False)


@functools.cache
def _sc_call(do_cnt):
  mesh = plsc.VectorSubcoreMesh(core_axis_name="c", subcore_axis_name="s",
                                num_cores=NC, num_subcores=NS)
  return pl.kernel(
      functools.partial(_sc_body, do_cnt),
      out_type=(
          jax.ShapeDtypeStruct((NC, NS, ENT_PT, CHH), jnp.float32),
          jax.ShapeDtypeStruct((NC, NS, USR_PT, CHH), jnp.float32),
          jax.ShapeDtypeStruct((NS, ENT_PT, 16), jnp.float32),
      ),
      mesh=mesh,
      compiler_params=pltpu.CompilerParams(needs_layout_passes=False,
                                           use_tc_tiling_on_sc=False),
      scratch_types=[
          pltpu.VMEM_SHARED((N_USR, CHH), jnp.float32),
          pltpu.VMEM_SHARED((N_ENT, 16), jnp.float32),
          pltpu.VMEM((C, CHH), jnp.float32),
          pltpu.VMEM((C, CHH), jnp.float32),
          pltpu.VMEM((C, CHH), jnp.float32),
          pltpu.VMEM((SUPE,), jnp.int32),
          pltpu.VMEM((SUPE,), jnp.int32),
          pltpu.VMEM((SUPE,), jnp.int32),
          pltpu.VMEM((SUPE,), jnp.int32),
          pltpu.VMEM((SCH, C), jnp.int32),
          pltpu.VMEM((SCH, C), jnp.int32),
          pltpu.VMEM((N_REL * CHH,), jnp.float32),
          pltpu.VMEM((C, 16), jnp.float32),
          pltpu.VMEM((ZROWS, CHH), jnp.float32),
          pltpu.VMEM((ZROWS, 16), jnp.float32),
          pltpu.SemaphoreType.DMA,
          pltpu.SemaphoreType.DMA,
          pltpu.SemaphoreType.DMA,
          pltpu.SemaphoreType.DMA,
          pltpu.SemaphoreType.DMA,
          pltpu.SemaphoreType.DMA,
          pltpu.SemaphoreType.DMA,
          pltpu.SemaphoreType.DMA,
          pltpu.SemaphoreType.DMA,
      ],
  )


_ENT_BLK = 2000
_USR_BLK = 2000


def _ent_dense_body(s0, s1, cnt, res_in, e0_o, e1_o, res_o):
  s = jnp.concatenate([s0[...], s1[...]], axis=1)
  d = s / jnp.maximum(cnt[...], 1.0)
  n = jnp.sqrt(jnp.sum(d * d, axis=1, keepdims=True))
  e = d / jnp.maximum(n, 1e-12)
  e0_o[...] = e[:, :CHH]
  e1_o[...] = e[:, CHH:]
  res_o[...] = res_in[...] + e


@jax.jit
def _ent_dense(s0, s1, cnt1, res_in):
  g = N_ENT // _ENT_BLK
  return pl.pallas_call(
      _ent_dense_body,
      grid=(g,),
      in_specs=[
          pl.BlockSpec((_ENT_BLK, CHH), lambda i: (i, 0)),
          pl.BlockSpec((_ENT_BLK, CHH), lambda i: (i, 0)),
          pl.BlockSpec((_ENT_BLK, 1), lambda i: (i, 0)),
          pl.BlockSpec((_ENT_BLK, CH), lambda i: (i, 0)),
      ],
      out_specs=[
          pl.BlockSpec((_ENT_BLK, CHH), lambda i: (i, 0)),
          pl.BlockSpec((_ENT_BLK, CHH), lambda i: (i, 0)),
          pl.BlockSpec((_ENT_BLK, CH), lambda i: (i, 0)),
      ],
      out_shape=[
          jax.ShapeDtypeStruct((N_ENT, CHH), jnp.float32),
          jax.ShapeDtypeStruct((N_ENT, CHH), jnp.float32),
          jax.ShapeDtypeStruct((N_ENT, CH), jnp.float32),
      ],
  )(s0, s1, cnt1, res_in)


def _usr_dense_body(s0, s1, up, res_in, lat, dw, u_o, res_o):
  z = lax.dot_general(up[...], lat[...], (((1,), (1,)), ((), ())),
                      preferred_element_type=jnp.float32)
  z = z - jnp.max(z, axis=1, keepdims=True)
  ez = jnp.exp(z)
  score = ez / jnp.sum(ez, axis=1, keepdims=True)
  gate = 1.0 + lax.dot_general(score, dw[...], (((1,), (0,)), ((), ())),
                               preferred_element_type=jnp.float32)
  x = jnp.concatenate([s0[...], s1[...]], axis=1) * gate
  n = jnp.sqrt(jnp.sum(x * x, axis=1, keepdims=True))
  u = x / jnp.maximum(n, 1e-12)
  u_o[...] = u
  res_o[...] = res_in[...] + u


@jax.jit
def _usr_dense(s0, s1, up, res_in, lat, dw):
  g = N_USR // _USR_BLK
  return pl.pallas_call(
      _usr_dense_body,
      grid=(g,),
      in_specs=[
          pl.BlockSpec((_USR_BLK, CHH), lambda i: (i, 0)),
          pl.BlockSpec((_USR_BLK, CHH), lambda i: (i, 0)),
          pl.BlockSpec((_USR_BLK, CH), lambda i: (i, 0)),
          pl.BlockSpec((_USR_BLK, CH), lambda i: (i, 0)),
          pl.BlockSpec((N_FAC, CH), lambda i: (0, 0)),
          pl.BlockSpec((N_FAC, CH), lambda i: (0, 0)),
      ],
      out_specs=[
          pl.BlockSpec((_USR_BLK, CH), lambda i: (i, 0)),
          pl.BlockSpec((_USR_BLK, CH), lambda i: (i, 0)),
      ],
      out_shape=[
          jax.ShapeDtypeStruct((N_USR, CH), jnp.float32),
          jax.ShapeDtypeStruct((N_USR, CH), jnp.float32),
      ],
  )(s0, s1, up, res_in, lat, dw)


def _cor_body(w_ref, da_ref, dw_o, cor_o):
  da = da_ref[...]
  m = jnp.max(da, axis=1, keepdims=True)
  ez = jnp.exp(da - m)
  sm = ez / jnp.sum(ez, axis=1, keepdims=True)
  dw_o[...] = lax.dot_general(sm, w_ref[...], (((1,), (0,)), ((), ())),
                              preferred_element_type=jnp.float32)
  nrm = jnp.sqrt(jnp.sum(da * da, axis=1, keepdims=True))
  nd = da / nrm
  gram = lax.dot_general(nd, nd, (((1,), (1,)), ((), ())),
                         preferred_element_type=jnp.float32)
  ii = lax.broadcasted_iota(jnp.int32, (N_FAC, N_FAC), 0)
  jj = lax.broadcasted_iota(jnp.int32, (N_FAC, N_FAC), 1)
  cor_o[...] = jnp.sum(jnp.where(ii < jj, gram * gram, 0.0),
                       keepdims=True).reshape(1, 1)


@jax.jit
def _cor_call(weight, disen_weight_att):
  return pl.pallas_call(
      _cor_body,
      out_shape=[
          jax.ShapeDtypeStruct((N_FAC, CH), jnp.float32),
          jax.ShapeDtypeStruct((1, 1), jnp.float32),
      ],
  )(weight, disen_weight_att)


def kernel(user_emb, entity_emb, latent_emb, edge_index, edge_type,
           interact_row, interact_col, interact_val, weight,
           disen_weight_att):
  head = edge_index[0]
  tail = edge_index[1]
  tail4 = jnp.pad(tail, (0, EPAD - E)).reshape(NS, ENSUP, SUPE)
  typ4 = jnp.pad(edge_type, (0, EPAD - E),
                 constant_values=N_REL).reshape(NS, ENSUP, SUPE)
  head4 = jnp.pad(head, (0, EPAD - E)).reshape(NS, ENSUP, SCH, C)
  col4 = jnp.pad(interact_col, (0, NPAD - NNZ)).reshape(NS, UNSUP, SUPE)
  val4 = lax.bitcast_convert_type(
      jnp.pad(interact_val, (0, NPAD - NNZ)),
      jnp.int32).reshape(NS, UNSUP, SUPE)
  row4 = jnp.pad(interact_row, (0, NPAD - NNZ)).reshape(NS, UNSUP, SCH, C)
  w12 = jnp.concatenate([weight, jnp.zeros((1, CH), jnp.float32)], axis=0)
  rel2 = w12.reshape(N_REL, NC, CHH).transpose(1, 0, 2).reshape(NC, N_REL * CHH)

  dw, cor = _cor_call(weight, disen_weight_att)

  ent_res, usr_res = entity_emb, user_emb
  u = user_emb
  e0 = entity_emb[:, :CHH]
  e1 = entity_emb[:, CHH:]
  cnt1 = None
  for hop in range(2):
    sc = _sc_call(hop == 0)
    ent_sums, usr_sums, cnt16 = sc(e0, e1, tail4, typ4, head4, col4,
                                   val4, row4, rel2)
    ent_sums = ent_sums.reshape(NC, N_ENT, CHH)
    usr_sums = usr_sums.reshape(NC, N_USR, CHH)
    if hop == 0:
      cnt1 = cnt16.reshape(N_ENT, 16)[:, :1]
    e0, e1, ent_res = _ent_dense(ent_sums[0], ent_sums[1], cnt1, ent_res)
    u, usr_res = _usr_dense(usr_sums[0], usr_sums[1], u, usr_res,
                            latent_emb, dw)
  return ent_res, usr_res, cor.reshape(())
